# baseline (speedup 1.0000x reference)
import functools

import jax
import jax.numpy as jnp
from jax import lax
from jax.experimental import pallas as pl
from jax.experimental.pallas import tpu as pltpu
from jax.experimental.pallas import tpu_sc as plsc

N = 4096
M = 12
S = 32
H = 128
C = 2
NM = N * M
MS = M * S
B = 128
BM = B * M
G = N // B

INV_G2_EDGE = (S / 8.0) ** 2
INV_G2_ANG = (S / 2.0) ** 2

NC = 2
NS = 16
NW = NC * NS
ROWS_W = NM // NW
CH = 128
NCH = ROWS_W // CH


def _sp(x):
    return jnp.maximum(x, 0.0) + jnp.log1p(jnp.exp(-jnp.abs(x)))


def _dot(a, w):
    return jnp.dot(a, w, preferred_element_type=jnp.float32)


def _leaky(x):
    return jnp.where(x >= 0, x, 0.01 * x)


def _rep_mat():
    r = lax.broadcasted_iota(jnp.int32, (M, MS), 0)
    c = lax.broadcasted_iota(jnp.int32, (M, MS), 1)
    return jnp.where(c // S == r, 1.0, 0.0)


def _pool_mat():
    r = lax.broadcasted_iota(jnp.int32, (MS, M), 0)
    c = lax.broadcasted_iota(jnp.int32, (MS, M), 1)
    return jnp.where(r // S == c, 1.0, 0.0)


def _sum_mat():
    r = lax.broadcasted_iota(jnp.int32, (MS, S), 0)
    c = lax.broadcasted_iota(jnp.int32, (MS, S), 1)
    return jnp.where(r % S == c, 1.0, 0.0)


def _ln(x, g, b):
    mu = jnp.mean(x, axis=-1, keepdims=True)
    xc = x - mu
    var = jnp.mean(xc * xc, axis=-1, keepdims=True)
    return xc / jnp.sqrt(var + 1e-5) * g + b


def _ang_basis_packed(afea, ftile, rep):
    return jnp.exp(-((_dot(afea, rep) - ftile) ** 2) * INV_G2_ANG)


def _proj(edge, gath_p, ang_p, wbig, bbig):
    y = _dot(jnp.concatenate([gath_p, ang_p, edge], axis=1), wbig) + bbig
    return y[:, 0:MS], y[:, MS:MS + M]


def _chunk_stats(x, pool):
    mu = _dot(x, pool) / S
    m2 = _dot(x * x, pool) / S
    return mu, m2 - mu * mu


def _conv_edge_blk(edge, gath_p, ang_p, wbig, bbig, repgb1, g2, b2, rep):
    lin_p, att = _proj(edge, gath_p, ang_p, wbig, bbig)
    att = _leaky(att)
    mx = jnp.max(att, axis=1, keepdims=True)
    ex = jnp.exp(att - mx)
    alpha = ex / jnp.sum(ex, axis=1, keepdims=True)
    mu, var = _chunk_stats(lin_p, _pool_mat())
    coef = alpha / jnp.sqrt(alpha * alpha * var + 1e-5)
    shift = -mu * coef
    hn = ((lin_p * _dot(coef, rep) + _dot(shift, rep))
          * repgb1[0:1] + repgb1[1:2])
    ssum = _dot(_sp(hn), _sum_mat())
    return _sp(_ln(edge + ssum, g2, b2))


def _conv_angle_blk(ang_p, edge, gath_p, wbig, bbig, repgb2, rep):
    lin_p, att = _proj(edge, gath_p, ang_p, wbig, bbig)
    s = ang_p + _dot(_leaky(att), rep) * lin_p
    mu, var = _chunk_stats(s, _pool_mat())
    coef = 1.0 / jnp.sqrt(var + 1e-5)
    shift = -mu * coef
    sn = ((s * _dot(coef, rep) + _dot(shift, rep))
          * repgb2[0:1] + repgb2[1:2])
    return _sp(sn)


def _k1_body(bond_ref, fe_ref, out_ref):
    out_ref[:] = jnp.exp(-((bond_ref[:] - fe_ref[:]) ** 2) * INV_G2_EDGE)


def _k2_body(edge_ref, gath_ref, afea_ref, ftile_ref,
             wbig_ref, bbig_ref, repgb1_ref, g2_ref, b2_ref,
             out_ref):
    rep = _rep_mat()
    ang_p = _ang_basis_packed(afea_ref[:], ftile_ref[:], rep)
    out_ref[:] = _conv_edge_blk(
        edge_ref[:], gath_ref[:], ang_p, wbig_ref[:], bbig_ref[:],
        repgb1_ref[:], g2_ref[:], b2_ref[:], rep)


def _k3_body(edge_ref, gath_ref, afea_ref, ftile_ref,
             cwbig_ref, cbbig_ref, crepgb_ref,
             ewbig_ref, ebbig_ref, erepgb_ref, eg2_ref, eb2_ref,
             angout_ref, edgeout_ref):
    rep = _rep_mat()
    edge = edge_ref[:]
    gath_p = gath_ref[:]
    ang0_p = _ang_basis_packed(afea_ref[:], ftile_ref[:], rep)
    ang1_p = _conv_angle_blk(ang0_p, edge, gath_p, cwbig_ref[:], cbbig_ref[:],
                             crepgb_ref[:], rep)
    angout_ref[:] = ang1_p
    edgeout_ref[:] = _conv_edge_blk(
        edge, gath_p, ang1_p, ewbig_ref[:], ebbig_ref[:],
        erepgb_ref[:], eg2_ref[:], eb2_ref[:], rep)


def _k4_body(edge_ref, gath_ref, angin_ref,
             cwbig_ref, cbbig_ref, crepgb_ref,
             ewbig_ref, ebbig_ref, erepgb_ref, eg2_ref, eb2_ref,
             expEW_ref, expEb_ref, expAW_ref, expAb_ref,
             bng_ref, bnb_ref, outW_ref, outb_ref,
             out_ref):
    rep = _rep_mat()
    edge = edge_ref[:]
    gath_p = gath_ref[:]
    ang2_p = _conv_angle_blk(angin_ref[:], edge, gath_p,
                             cwbig_ref[:], cbbig_ref[:],
                             crepgb_ref[:], rep)
    edge3 = _conv_edge_blk(edge, gath_p, ang2_p, ewbig_ref[:], ebbig_ref[:],
                           erepgb_ref[:], eg2_ref[:], eb2_ref[:], rep)
    pe = _sp(_dot(edge3, expEW_ref[:]) + expEb_ref[:])
    acc = jnp.zeros_like(pe)
    for k in range(M):
        acc = acc + _sp(_dot(ang2_p[:, k * S:(k + 1) * S], expAW_ref[:])
                        + expAb_ref[:])
    pa = _sp(acc)
    r = lax.broadcasted_iota(jnp.int32, (B, BM), 1)
    bi = lax.broadcasted_iota(jnp.int32, (B, BM), 0)
    seg = jnp.where(r // M == bi, 1.0, 0.0)
    e = _dot(seg, pe)
    a = _dot(seg, pa)
    crys = jnp.concatenate([e, a], axis=1)
    crys = _sp(_ln(crys, bng_ref[:], bnb_ref[:]))
    out_ref[:] = _dot(crys, outW_ref[:]) + outb_ref[:]


@functools.lru_cache(maxsize=1)
def _gather_call():
    mesh = plsc.VectorSubcoreMesh(core_axis_name="c", subcore_axis_name="s")

    @functools.partial(
        pl.kernel,
        mesh=mesh,
        out_type=jax.ShapeDtypeStruct((NM, MS), jnp.float32),
        scratch_types=[
            pltpu.VMEM((ROWS_W,), jnp.int32),
            pltpu.VMEM((CH, MS), jnp.float32),
            pltpu.VMEM((CH, MS), jnp.float32),
            pltpu.SemaphoreType.DMA,
            pltpu.SemaphoreType.DMA,
        ],
    )
    def _sc_gather(table_hbm, idx_hbm, out_hbm, idx_v, rows0, rows1,
                   sem0, sem1):
        wid = lax.axis_index("s") * NC + lax.axis_index("c")
        base = wid * ROWS_W
        pltpu.sync_copy(idx_hbm.at[pl.ds(base, ROWS_W)], idx_v)
        rows = (rows0, rows1)
        sems = (sem0, sem1)

        def start(c):
            return pltpu.async_copy(
                table_hbm.at[idx_v.at[pl.ds(c * CH, CH)]],
                rows[c % 2], sems[c % 2])

        cp = start(0)
        for c in range(NCH):
            nxt = start(c + 1) if c + 1 < NCH else None
            cp.wait()
            pltpu.sync_copy(rows[c % 2], out_hbm.at[pl.ds(base + c * CH, CH)])
            cp = nxt

    return _sc_gather


def _gather(table, idx):
    return _gather_call()(table, idx)


def _spec(shape, blocked=True):
    if blocked:
        return pl.BlockSpec(shape, lambda g: (g, 0))
    return pl.BlockSpec(shape, lambda g: (0, 0))


def kernel(bond_fea, angle_fea, species, nbr_idx, crys_idx,
           ce_lin_W, ce_lin_b, ce_att_W, ce_att_b, ce_g1, ce_b1, ce_g2, ce_b2,
           ca_lin_W, ca_lin_b, ca_att_W, ca_att_b, ca_g2, ca_b2,
           expE_W, expE_b, expA_W, expA_b, bn_g, bn_b, out_W, out_b):
    f32 = jnp.float32
    bond2 = bond_fea.reshape(NM, 1).astype(f32)
    afea2 = angle_fea.reshape(NM, M).astype(f32)
    nbrf = nbr_idx.reshape(NM).astype(jnp.int32)
    f_edge = jnp.linspace(0.0, 8.0, S, dtype=f32).reshape(1, S)
    f_ang = jnp.linspace(-1.0, 1.0, S, dtype=f32).reshape(1, S)
    ftile = jnp.tile(f_ang, (1, M))
    eye = jnp.eye(M, dtype=f32)

    def mk_w(lw, lb, aw, ab, g2, b2):
        wbig = jnp.concatenate([
            jnp.concatenate([jnp.kron(eye, lw[S:2 * S]),
                             jnp.kron(eye, aw[S:2 * S])], axis=1),
            jnp.concatenate([jnp.kron(eye, lw[2 * S:3 * S]),
                             jnp.kron(eye, aw[2 * S:3 * S])], axis=1),
            jnp.concatenate([jnp.tile(lw[0:S], (1, M)),
                             jnp.tile(aw[0:S], (1, M))], axis=1),
        ], axis=0)
        bbig = jnp.concatenate([jnp.tile(lb.reshape(1, S), (1, M)),
                                jnp.tile(ab.reshape(1, 1), (1, M))], axis=1)
        repgb = jnp.concatenate(
            [jnp.tile(g2.reshape(1, S), (1, M)),
             jnp.tile(b2.reshape(1, S), (1, M))], axis=0)
        return wbig, bbig, repgb

    def ce_w(i):
        wbig, bbig, repgb = mk_w(ce_lin_W[i], ce_lin_b[i],
                                 ce_att_W[i], ce_att_b[i],
                                 ce_g1[i], ce_b1[i])
        return (wbig, bbig, repgb,
                ce_g2[i].reshape(1, S), ce_b2[i].reshape(1, S))

    def ca_w(i):
        return mk_w(ca_lin_W[i], ca_lin_b[i],
                    ca_att_W[i], ca_att_b[i],
                    ca_g2[i], ca_b2[i])

    wspec_wbig = _spec((2 * MS + S, MS + M), blocked=False)
    wspec_bbig = _spec((1, MS + M), blocked=False)
    wspec_repgb = _spec((2, MS), blocked=False)
    wspec_t = _spec((1, MS), blocked=False)
    wspec_row = _spec((1, S), blocked=False)
    ce_specs = [wspec_wbig, wspec_bbig, wspec_repgb, wspec_row, wspec_row]
    ca_specs = [wspec_wbig, wspec_bbig, wspec_repgb]

    edge0 = pl.pallas_call(
        _k1_body,
        grid=(G,),
        in_specs=[_spec((BM, 1)), wspec_row],
        out_specs=_spec((BM, S)),
        out_shape=jax.ShapeDtypeStruct((NM, S), f32),
    )(bond2, f_edge)

    gath0 = _gather(edge0.reshape(N, MS), nbrf)

    edge1 = pl.pallas_call(
        _k2_body,
        grid=(G,),
        in_specs=[_spec((BM, S)), _spec((BM, MS)), _spec((BM, M)),
                  wspec_t] + ce_specs,
        out_specs=_spec((BM, S)),
        out_shape=jax.ShapeDtypeStruct((NM, S), f32),
    )(edge0, gath0, afea2, ftile, *ce_w(0))

    gath1 = _gather(edge1.reshape(N, MS), nbrf)

    angle1, edge2 = pl.pallas_call(
        _k3_body,
        grid=(G,),
        in_specs=[_spec((BM, S)), _spec((BM, MS)), _spec((BM, M)),
                  wspec_t] + ca_specs + ce_specs,
        out_specs=[_spec((BM, MS)), _spec((BM, S))],
        out_shape=[jax.ShapeDtypeStruct((NM, MS), f32),
                   jax.ShapeDtypeStruct((NM, S), f32)],
    )(edge1, gath1, afea2, ftile, *ca_w(0), *ce_w(1))

    gath2 = _gather(edge2.reshape(N, MS), nbrf)

    out = pl.pallas_call(
        _k4_body,
        grid=(G,),
        in_specs=[_spec((BM, S)), _spec((BM, MS)), _spec((BM, MS))]
                 + ca_specs + ce_specs
                 + [_spec((S, H), blocked=False), _spec((1, H), blocked=False),
                    _spec((S, H), blocked=False), _spec((1, H), blocked=False),
                    _spec((1, 2 * H), blocked=False),
                    _spec((1, 2 * H), blocked=False),
                    _spec((2 * H, C), blocked=False),
                    _spec((1, C), blocked=False)],
        out_specs=_spec((B, C)),
        out_shape=jax.ShapeDtypeStruct((N, C), f32),
    )(edge2, gath2, angle1, *ca_w(1), *ce_w(2),
      expE_W, expE_b.reshape(1, H), expA_W, expA_b.reshape(1, H),
      bn_g.reshape(1, 2 * H), bn_b.reshape(1, 2 * H),
      out_W, out_b.reshape(1, C))

    return out


# speedup vs baseline: 1.0101x; 1.0101x over previous
import functools

import jax
import jax.numpy as jnp
from jax import lax
from jax.experimental import pallas as pl
from jax.experimental.pallas import tpu as pltpu
from jax.experimental.pallas import tpu_sc as plsc

N = 4096
M = 12
S = 32
H = 128
C = 2
NM = N * M
MS = M * S
B = 128
BM = B * M
G = N // B

INV_G2_EDGE = (S / 8.0) ** 2
INV_G2_ANG = (S / 2.0) ** 2

NC = 2
NS = 16
NW = NC * NS
ROWS_W = NM // NW
CH = 128
NCH = ROWS_W // CH


def _sp(x):
    return jnp.maximum(x, 0.0) + jnp.log1p(jnp.exp(-jnp.abs(x)))


def _dot(a, w):
    return jnp.dot(a, w, preferred_element_type=jnp.float32)


def _leaky(x):
    return jnp.where(x >= 0, x, 0.01 * x)


def _rep_mat():
    r = lax.broadcasted_iota(jnp.int32, (M, MS), 0)
    c = lax.broadcasted_iota(jnp.int32, (M, MS), 1)
    return jnp.where(c // S == r, 1.0, 0.0)


def _pool_mat():
    r = lax.broadcasted_iota(jnp.int32, (MS, M), 0)
    c = lax.broadcasted_iota(jnp.int32, (MS, M), 1)
    return jnp.where(r // S == c, 1.0, 0.0)


def _sum_mat():
    r = lax.broadcasted_iota(jnp.int32, (MS, S), 0)
    c = lax.broadcasted_iota(jnp.int32, (MS, S), 1)
    return jnp.where(r % S == c, 1.0, 0.0)


def _ln(x, g, b):
    mu = jnp.mean(x, axis=-1, keepdims=True)
    xc = x - mu
    var = jnp.mean(xc * xc, axis=-1, keepdims=True)
    return xc / jnp.sqrt(var + 1e-5) * g + b


def _ang_basis_packed(afea, ftile, rep):
    return jnp.exp(-((_dot(afea, rep) - ftile) ** 2) * INV_G2_ANG)


def _proj(edge, gath_p, ang_p, wbig, bbig):
    y = _dot(jnp.concatenate([gath_p, ang_p, edge], axis=1), wbig) + bbig
    return y[:, 0:MS], y[:, MS:MS + M]


def _chunk_center(x, pool, rep):
    mu = _dot(x, pool) / S
    d = x - _dot(mu, rep)
    var = _dot(d * d, pool) / S
    return d, var


def _conv_edge_blk(edge, gath_p, ang_p, wbig, bbig, repgb1, g2, b2, rep):
    lin_p, att = _proj(edge, gath_p, ang_p, wbig, bbig)
    att = _leaky(att)
    mx = jnp.max(att, axis=1, keepdims=True)
    ex = jnp.exp(att - mx)
    alpha = ex / jnp.sum(ex, axis=1, keepdims=True)
    d, var = _chunk_center(lin_p, _pool_mat(), rep)
    coef = alpha / jnp.sqrt(alpha * alpha * var + 1e-5)
    hn = d * _dot(coef, rep) * repgb1[0:1] + repgb1[1:2]
    ssum = _dot(_sp(hn), _sum_mat())
    return _sp(_ln(edge + ssum, g2, b2))


def _conv_angle_blk(ang_p, edge, gath_p, wbig, bbig, repgb2, rep):
    lin_p, att = _proj(edge, gath_p, ang_p, wbig, bbig)
    s = ang_p + _dot(_leaky(att), rep) * lin_p
    d, var = _chunk_center(s, _pool_mat(), rep)
    coef = 1.0 / jnp.sqrt(var + 1e-5)
    sn = d * _dot(coef, rep) * repgb2[0:1] + repgb2[1:2]
    return _sp(sn)


def _k1_body(bond_ref, fe_ref, out_ref):
    out_ref[:] = jnp.exp(-((bond_ref[:] - fe_ref[:]) ** 2) * INV_G2_EDGE)


def _k2_body(edge_ref, gath_ref, afea_ref, ftile_ref,
             wbig_ref, bbig_ref, repgb1_ref, g2_ref, b2_ref,
             out_ref):
    rep = _rep_mat()
    ang_p = _ang_basis_packed(afea_ref[:], ftile_ref[:], rep)
    out_ref[:] = _conv_edge_blk(
        edge_ref[:], gath_ref[:], ang_p, wbig_ref[:], bbig_ref[:],
        repgb1_ref[:], g2_ref[:], b2_ref[:], rep)


def _k3_body(edge_ref, gath_ref, afea_ref, ftile_ref,
             cwbig_ref, cbbig_ref, crepgb_ref,
             ewbig_ref, ebbig_ref, erepgb_ref, eg2_ref, eb2_ref,
             angout_ref, edgeout_ref):
    rep = _rep_mat()
    edge = edge_ref[:]
    gath_p = gath_ref[:]
    ang0_p = _ang_basis_packed(afea_ref[:], ftile_ref[:], rep)
    ang1_p = _conv_angle_blk(ang0_p, edge, gath_p, cwbig_ref[:], cbbig_ref[:],
                             crepgb_ref[:], rep)
    angout_ref[:] = ang1_p
    edgeout_ref[:] = _conv_edge_blk(
        edge, gath_p, ang1_p, ewbig_ref[:], ebbig_ref[:],
        erepgb_ref[:], eg2_ref[:], eb2_ref[:], rep)


def _k4_body(edge_ref, gath_ref, angin_ref,
             cwbig_ref, cbbig_ref, crepgb_ref,
             ewbig_ref, ebbig_ref, erepgb_ref, eg2_ref, eb2_ref,
             expEW_ref, expEb_ref, expAW_ref, expAb_ref,
             bng_ref, bnb_ref, outW_ref, outb_ref,
             out_ref):
    rep = _rep_mat()
    edge = edge_ref[:]
    gath_p = gath_ref[:]
    ang2_p = _conv_angle_blk(angin_ref[:], edge, gath_p,
                             cwbig_ref[:], cbbig_ref[:],
                             crepgb_ref[:], rep)
    edge3 = _conv_edge_blk(edge, gath_p, ang2_p, ewbig_ref[:], ebbig_ref[:],
                           erepgb_ref[:], eg2_ref[:], eb2_ref[:], rep)
    pe = _sp(_dot(edge3, expEW_ref[:]) + expEb_ref[:])
    acc = jnp.zeros_like(pe)
    for k in range(M):
        acc = acc + _sp(_dot(ang2_p[:, k * S:(k + 1) * S], expAW_ref[:])
                        + expAb_ref[:])
    pa = _sp(acc)
    r = lax.broadcasted_iota(jnp.int32, (B, BM), 1)
    bi = lax.broadcasted_iota(jnp.int32, (B, BM), 0)
    seg = jnp.where(r // M == bi, 1.0, 0.0)
    e = _dot(seg, pe)
    a = _dot(seg, pa)
    crys = jnp.concatenate([e, a], axis=1)
    crys = _sp(_ln(crys, bng_ref[:], bnb_ref[:]))
    out_ref[:] = _dot(crys, outW_ref[:]) + outb_ref[:]


@functools.lru_cache(maxsize=1)
def _gather_call():
    mesh = plsc.VectorSubcoreMesh(core_axis_name="c", subcore_axis_name="s")

    @functools.partial(
        pl.kernel,
        mesh=mesh,
        out_type=jax.ShapeDtypeStruct((NM, MS), jnp.float32),
        scratch_types=[
            pltpu.VMEM((ROWS_W,), jnp.int32),
            pltpu.VMEM((CH, MS), jnp.float32),
            pltpu.VMEM((CH, MS), jnp.float32),
            pltpu.SemaphoreType.DMA,
            pltpu.SemaphoreType.DMA,
        ],
    )
    def _sc_gather(table_hbm, idx_hbm, out_hbm, idx_v, rows0, rows1,
                   sem0, sem1):
        wid = lax.axis_index("s") * NC + lax.axis_index("c")
        base = wid * ROWS_W
        pltpu.sync_copy(idx_hbm.at[pl.ds(base, ROWS_W)], idx_v)
        rows = (rows0, rows1)
        sems = (sem0, sem1)

        def start(c):
            return pltpu.async_copy(
                table_hbm.at[idx_v.at[pl.ds(c * CH, CH)]],
                rows[c % 2], sems[c % 2])

        cp = start(0)
        for c in range(NCH):
            nxt = start(c + 1) if c + 1 < NCH else None
            cp.wait()
            pltpu.sync_copy(rows[c % 2], out_hbm.at[pl.ds(base + c * CH, CH)])
            cp = nxt

    return _sc_gather


def _gather(table, idx):
    return _gather_call()(table, idx)


def _spec(shape, blocked=True):
    if blocked:
        return pl.BlockSpec(shape, lambda g: (g, 0))
    return pl.BlockSpec(shape, lambda g: (0, 0))


def kernel(bond_fea, angle_fea, species, nbr_idx, crys_idx,
           ce_lin_W, ce_lin_b, ce_att_W, ce_att_b, ce_g1, ce_b1, ce_g2, ce_b2,
           ca_lin_W, ca_lin_b, ca_att_W, ca_att_b, ca_g2, ca_b2,
           expE_W, expE_b, expA_W, expA_b, bn_g, bn_b, out_W, out_b):
    f32 = jnp.float32
    bond2 = bond_fea.reshape(NM, 1).astype(f32)
    afea2 = angle_fea.reshape(NM, M).astype(f32)
    nbrf = nbr_idx.reshape(NM).astype(jnp.int32)
    f_edge = jnp.linspace(0.0, 8.0, S, dtype=f32).reshape(1, S)
    f_ang = jnp.linspace(-1.0, 1.0, S, dtype=f32).reshape(1, S)
    ftile = jnp.tile(f_ang, (1, M))
    eye = jnp.eye(M, dtype=f32)

    def mk_w(lw, lb, aw, ab, g2, b2):
        wbig = jnp.concatenate([
            jnp.concatenate([jnp.kron(eye, lw[S:2 * S]),
                             jnp.kron(eye, aw[S:2 * S])], axis=1),
            jnp.concatenate([jnp.kron(eye, lw[2 * S:3 * S]),
                             jnp.kron(eye, aw[2 * S:3 * S])], axis=1),
            jnp.concatenate([jnp.tile(lw[0:S], (1, M)),
                             jnp.tile(aw[0:S], (1, M))], axis=1),
        ], axis=0)
        bbig = jnp.concatenate([jnp.tile(lb.reshape(1, S), (1, M)),
                                jnp.tile(ab.reshape(1, 1), (1, M))], axis=1)
        repgb = jnp.concatenate(
            [jnp.tile(g2.reshape(1, S), (1, M)),
             jnp.tile(b2.reshape(1, S), (1, M))], axis=0)
        return wbig, bbig, repgb

    def ce_w(i):
        wbig, bbig, repgb = mk_w(ce_lin_W[i], ce_lin_b[i],
                                 ce_att_W[i], ce_att_b[i],
                                 ce_g1[i], ce_b1[i])
        return (wbig, bbig, repgb,
                ce_g2[i].reshape(1, S), ce_b2[i].reshape(1, S))

    def ca_w(i):
        return mk_w(ca_lin_W[i], ca_lin_b[i],
                    ca_att_W[i], ca_att_b[i],
                    ca_g2[i], ca_b2[i])

    wspec_wbig = _spec((2 * MS + S, MS + M), blocked=False)
    wspec_bbig = _spec((1, MS + M), blocked=False)
    wspec_repgb = _spec((2, MS), blocked=False)
    wspec_t = _spec((1, MS), blocked=False)
    wspec_row = _spec((1, S), blocked=False)
    ce_specs = [wspec_wbig, wspec_bbig, wspec_repgb, wspec_row, wspec_row]
    ca_specs = [wspec_wbig, wspec_bbig, wspec_repgb]

    edge0 = pl.pallas_call(
        _k1_body,
        grid=(G,),
        in_specs=[_spec((BM, 1)), wspec_row],
        out_specs=_spec((BM, S)),
        out_shape=jax.ShapeDtypeStruct((NM, S), f32),
    )(bond2, f_edge)

    gath0 = _gather(edge0.reshape(N, MS), nbrf)

    edge1 = pl.pallas_call(
        _k2_body,
        grid=(G,),
        in_specs=[_spec((BM, S)), _spec((BM, MS)), _spec((BM, M)),
                  wspec_t] + ce_specs,
        out_specs=_spec((BM, S)),
        out_shape=jax.ShapeDtypeStruct((NM, S), f32),
    )(edge0, gath0, afea2, ftile, *ce_w(0))

    gath1 = _gather(edge1.reshape(N, MS), nbrf)

    angle1, edge2 = pl.pallas_call(
        _k3_body,
        grid=(G,),
        in_specs=[_spec((BM, S)), _spec((BM, MS)), _spec((BM, M)),
                  wspec_t] + ca_specs + ce_specs,
        out_specs=[_spec((BM, MS)), _spec((BM, S))],
        out_shape=[jax.ShapeDtypeStruct((NM, MS), f32),
                   jax.ShapeDtypeStruct((NM, S), f32)],
    )(edge1, gath1, afea2, ftile, *ca_w(0), *ce_w(1))

    gath2 = _gather(edge2.reshape(N, MS), nbrf)

    out = pl.pallas_call(
        _k4_body,
        grid=(G,),
        in_specs=[_spec((BM, S)), _spec((BM, MS)), _spec((BM, MS))]
                 + ca_specs + ce_specs
                 + [_spec((S, H), blocked=False), _spec((1, H), blocked=False),
                    _spec((S, H), blocked=False), _spec((1, H), blocked=False),
                    _spec((1, 2 * H), blocked=False),
                    _spec((1, 2 * H), blocked=False),
                    _spec((2 * H, C), blocked=False),
                    _spec((1, C), blocked=False)],
        out_specs=_spec((B, C)),
        out_shape=jax.ShapeDtypeStruct((N, C), f32),
    )(edge2, gath2, angle1, *ca_w(1), *ce_w(2),
      expE_W, expE_b.reshape(1, H), expA_W, expA_b.reshape(1, H),
      bn_g.reshape(1, 2 * H), bn_b.reshape(1, 2 * H),
      out_W, out_b.reshape(1, C))

    return out
